# baseline (speedup 1.0000x reference)
import functools

import jax
import jax.numpy as jnp
from jax import lax
from jax.experimental import pallas as pl
from jax.experimental.pallas import tpu as pltpu
from jax.experimental.pallas import tpu_sc as plsc

VOCAB = 1000000
DIM = 64
BATCH = 4096
HIST = 200

NC, NS = 2, 16
NW = NC * NS
TOTAL = BATCH * HIST
B_PER_W = TOTAL // NW
CHUNK = 128
N_CHUNKS = B_PER_W // CHUNK


def _embed_body(idx_hbm, table_hbm, out_hbm, idx_v, rows0, rows1, sem_g0,
                sem_g1, sem_o0, sem_o1):
    wid = lax.axis_index("s") * NC + lax.axis_index("c")
    base = wid * B_PER_W
    pltpu.sync_copy(idx_hbm.at[pl.ds(base, B_PER_W)], idx_v)

    rows = (rows0, rows1)
    sem_g = (sem_g0, sem_g1)
    sem_o = (sem_o0, sem_o1)

    def start_gather(j, b):
        pltpu.async_copy(table_hbm.at[idx_v.at[pl.ds(j * CHUNK, CHUNK)]],
                         rows[b], sem_g[b])

    def start_out(j, b):
        pltpu.async_copy(rows[b], out_hbm.at[pl.ds(base + j * CHUNK, CHUNK)],
                         sem_o[b])

    def wait_gather(b):
        pltpu.make_async_copy(out_hbm.at[pl.ds(0, CHUNK)], rows[b],
                              sem_g[b]).wait()

    def wait_out(b):
        pltpu.make_async_copy(rows[b], out_hbm.at[pl.ds(base, CHUNK)],
                              sem_o[b]).wait()

    start_gather(0, 0)
    start_gather(1, 1)

    def step(i, _):
        j0 = 2 * i
        wait_gather(0)
        start_out(j0, 0)
        wait_gather(1)
        start_out(j0 + 1, 1)
        wait_out(0)
        start_gather(j0 + 2, 0)
        wait_out(1)
        start_gather(j0 + 3, 1)
        return _

    lax.fori_loop(0, N_CHUNKS // 2 - 1, step, 0)

    j0 = N_CHUNKS - 2
    wait_gather(0)
    start_out(j0, 0)
    wait_gather(1)
    start_out(j0 + 1, 1)
    wait_out(0)
    wait_out(1)


@jax.jit
def _embed(idx_flat, embeddings):
    mesh = plsc.VectorSubcoreMesh(core_axis_name="c", subcore_axis_name="s")
    return pl.kernel(
        _embed_body,
        out_type=jax.ShapeDtypeStruct((TOTAL, DIM), jnp.float32),
        mesh=mesh,
        compiler_params=pltpu.CompilerParams(use_tc_tiling_on_sc=False),
        scratch_types=[
            pltpu.VMEM((B_PER_W,), jnp.int32),
            pltpu.VMEM((CHUNK, DIM), jnp.float32),
            pltpu.VMEM((CHUNK, DIM), jnp.float32),
            pltpu.SemaphoreType.DMA,
            pltpu.SemaphoreType.DMA,
            pltpu.SemaphoreType.DMA,
            pltpu.SemaphoreType.DMA,
        ],
    )(idx_flat, embeddings)


def kernel(inp, embeddings):
    idx_flat = inp.reshape(TOTAL).astype(jnp.int32)
    out = _embed(idx_flat, embeddings)
    return out.reshape(BATCH, HIST, DIM)
